# baseline (speedup 1.0000x reference)
import jax
import jax.numpy as jnp
from jax import lax
from jax.experimental import pallas as pl
from jax.experimental.pallas import tpu as pltpu
from jax.experimental.pallas import tpu_sc as plsc

MIN_WINDOW_SIZE = 33
MAX_WINDOW_SIZE = 99

_N = 5000
_NPAD = 5120
_NC, _NS = 2, 16
_NW = _NC * _NS
_PER_W = _NPAD // _NW
_GROUPS = _PER_W // 16
_ROWW = _NPAD // 4
_NBUF = 2
_NP = 26 + 4 * _NBUF


def _half_kernel(boxes_ref, half_ref):
    wh = boxes_ref[:, 2:4]
    mx = jnp.max(wh, axis=1)
    mn = jnp.min(wh, axis=1)
    scale = jnp.sqrt(mx / mn)
    window = (MIN_WINDOW_SIZE * scale).astype(jnp.int32)
    window = jnp.clip(window, MIN_WINDOW_SIZE, MAX_WINDOW_SIZE)
    half_ref[0:1, :_N] = (window // 2).reshape(1, _N)
    half_ref[0:1, _N:] = jnp.zeros((1, _NPAD - _N), jnp.int32)


def _compute_half(boxes):
    return pl.pallas_call(
        _half_kernel,
        out_shape=jax.ShapeDtypeStruct((1, _NPAD), jnp.int32),
    )(boxes)


def _sc_band(half_hbm, out_hbm, half_v, buf, sem0, sem1):
    cid = lax.axis_index("c")
    sid = lax.axis_index("s")
    wid = sid * _NC + cid
    base = wid * _PER_W
    pltpu.sync_copy(half_hbm.at[0], half_v)

    def zbody(k, _):
        for u in range(16):
            buf[pl.ds((k * 16 + u) * 16, 16)] = jnp.zeros((16,), jnp.int32)
        return 0
    lax.fori_loop(0, _NBUF * 16 * _ROWW // 256, zbody, 0)

    lane = lax.iota(jnp.int32, 16)
    sems = (sem0, sem1)
    handles = [None] * _GROUPS

    for g in range(_GROUPS):
        slot = g % _NBUF
        if g >= _NBUF:
            for cp, i_row in handles[g - _NBUF]:
                @pl.when(i_row < _N)
                def _w(cp=cp):
                    cp.wait()

        hv = half_v[pl.ds(base + g * 16, 16)]
        rows = base + g * 16 + lane
        s0 = jnp.maximum(((rows - 49) >> 2) - 4 * _NBUF, 0)
        lanebase = (slot * 16 + lane) * _ROWW

        def pbody(p, _, s0=s0, rows=rows, hv=hv, lanebase=lanebase):
            wp = jnp.minimum(s0 + p, _ROWW - 1)
            b = wp << 2
            acc = jnp.zeros((16,), jnp.int32)
            for m in range(4):
                inb = jnp.abs(b + m - rows) <= hv
                acc = acc + jnp.where(inb, jnp.int32(1 << (8 * m)),
                                      jnp.int32(0))
            plsc.store_scatter(buf, [lanebase + wp], acc)
            return 0
        lax.fori_loop(0, _NP, pbody, 0)

        glist = []
        for l in range(16):
            i_row = base + g * 16 + l
            src = buf.at[pl.ds((slot * 16 + l) * _ROWW, _ROWW)]
            cp = pltpu.make_async_copy(src, out_hbm.at[i_row], sems[slot])

            @pl.when(i_row < _N)
            def _s(cp=cp):
                cp.start()
            glist.append((cp, i_row))
        handles[g] = glist

    for g in range(_GROUPS - _NBUF, _GROUPS):
        for cp, i_row in handles[g]:
            @pl.when(i_row < _N)
            def _w(cp=cp):
                cp.wait()


@jax.jit
def kernel(boxes):
    half = _compute_half(boxes)
    mesh = plsc.VectorSubcoreMesh(core_axis_name="c", subcore_axis_name="s")
    sc = pl.kernel(
        _sc_band,
        out_type=jax.ShapeDtypeStruct((_N, _ROWW), jnp.int32),
        mesh=mesh,
        compiler_params=pltpu.CompilerParams(needs_layout_passes=False),
        scratch_types=[
            pltpu.VMEM((_NPAD,), jnp.int32),
            pltpu.VMEM((_NBUF * 16 * _ROWW,), jnp.int32),
            pltpu.SemaphoreType.DMA,
            pltpu.SemaphoreType.DMA,
        ],
    )
    words = sc(half)
    mask_bytes = lax.bitcast_convert_type(words, jnp.uint8)
    mask_bytes = mask_bytes.reshape(_N, _NPAD)[:, :_N]
    return mask_bytes.astype(jnp.bool_)
